# baseline (speedup 1.0000x reference)
import jax
import jax.numpy as jnp
from jax.experimental import pallas as pl
from jax.experimental.pallas import tpu as pltpu

_K = 64


def _ea_fused_kernel(x_ref, w1_ref, b1_ref, w0_ref, wc2_ref, scale_ref,
                     shift_ref, o_ref, b1c_sc, bmat_sc, shiftc_sc):
    @pl.when(pl.program_id(0) == 0)
    def _prep():
        b1c_sc[...] = b1_ref[...].T
        shiftc_sc[...] = shift_ref[...].T
        wc2f = scale_ref[...].T * wc2_ref[...]
        bmat_sc[...] = jnp.dot(wc2f, w0_ref[...].T,
                               precision=jax.lax.Precision.HIGHEST,
                               preferred_element_type=jnp.float32)

    for t in range(x_ref.shape[0]):
        x = x_ref[t]
        y = jnp.dot(w1_ref[...], x,
                    preferred_element_type=jnp.float32) + b1c_sc[...]
        s = jnp.dot(w0_ref[...], y,
                    preferred_element_type=jnp.float32)
        m = jnp.max(s, axis=-1, keepdims=True)
        e = jnp.exp(s - m)
        l = jnp.sum(e, axis=-1, keepdims=True)
        attn = e * pl.reciprocal(l, approx=True)
        denom = 1e-9 + jnp.sum(attn, axis=0, keepdims=True)
        attn = attn * pl.reciprocal(denom, approx=True)
        z = jnp.dot(bmat_sc[...], attn,
                    preferred_element_type=jnp.float32) + shiftc_sc[...]
        o_ref[t] = jnp.maximum(z + x, 0.0).astype(o_ref.dtype)


@jax.jit
def _external_attention(x_nchw, w1, b1, w0, wc2, bn_scale, bn_shift):
    b, c, h, w = x_nchw.shape
    n = h * w

    x = x_nchw.reshape(b, c, n)
    b1r = b1.reshape(1, c)
    scaler = bn_scale.reshape(1, c)
    shiftr = bn_shift.reshape(1, c)

    out = pl.pallas_call(
        _ea_fused_kernel,
        out_shape=jax.ShapeDtypeStruct((b, c, n), x.dtype),
        grid=(b // 2,),
        in_specs=[
            pl.BlockSpec((2, c, n), lambda i: (i, 0, 0)),
            pl.BlockSpec((c, c), lambda i: (0, 0),
                         pipeline_mode=pl.Buffered(buffer_count=1)),
            pl.BlockSpec((1, c), lambda i: (0, 0),
                         pipeline_mode=pl.Buffered(buffer_count=1)),
            pl.BlockSpec((_K, c), lambda i: (0, 0),
                         pipeline_mode=pl.Buffered(buffer_count=1)),
            pl.BlockSpec((c, c), lambda i: (0, 0),
                         pipeline_mode=pl.Buffered(buffer_count=1)),
            pl.BlockSpec((1, c), lambda i: (0, 0),
                         pipeline_mode=pl.Buffered(buffer_count=1)),
            pl.BlockSpec((1, c), lambda i: (0, 0),
                         pipeline_mode=pl.Buffered(buffer_count=1)),
        ],
        out_specs=pl.BlockSpec((2, c, n), lambda i: (i, 0, 0)),
        scratch_shapes=[
            pltpu.VMEM((c, 1), jnp.float32),
            pltpu.VMEM((c, _K), jnp.float32),
            pltpu.VMEM((c, 1), jnp.float32),
        ],
        compiler_params=pltpu.CompilerParams(
            dimension_semantics=("arbitrary",)),
    )(x, w1, b1r, w0, wc2, scaler, shiftr)

    return out.reshape(b, c, h, w)


def kernel(x, w1, b1, w0, wc2, bn_scale, bn_shift):
    return _external_attention(x, w1, b1, w0, wc2, bn_scale, bn_shift)


# speedup vs baseline: 1.0371x; 1.0371x over previous
import jax
import jax.numpy as jnp
from jax.experimental import pallas as pl
from jax.experimental.pallas import tpu as pltpu

_K = 64


def _ea_fused_kernel(x_ref, w1_ref, b1_ref, w0_ref, wc2_ref, scale_ref,
                     shift_ref, o_ref, b1c_sc, bmat_sc, shiftc_sc):
    @pl.when(pl.program_id(0) == 0)
    def _prep():
        b1c_sc[...] = b1_ref[...].T
        shiftc_sc[...] = shift_ref[...].T
        wc2f = scale_ref[...].T * wc2_ref[...]
        bmat_sc[...] = jnp.dot(wc2f, w0_ref[...].T,
                               precision=jax.lax.Precision.HIGHEST,
                               preferred_element_type=jnp.float32)

    for t in range(x_ref.shape[0]):
        x = x_ref[t]
        y = jnp.dot(w1_ref[...], x,
                    preferred_element_type=jnp.float32) + b1c_sc[...]
        s = jnp.dot(w0_ref[...], y,
                    preferred_element_type=jnp.float32)
        m = jnp.max(s, axis=-1, keepdims=True)
        e = jnp.exp(s - m)
        l = jnp.sum(e, axis=-1, keepdims=True)
        attn = e * pl.reciprocal(l, approx=True)
        denom = 1e-9 + jnp.sum(attn, axis=0, keepdims=True)
        attn = attn * pl.reciprocal(denom, approx=True)
        z = jnp.dot(bmat_sc[...], attn,
                    preferred_element_type=jnp.float32) + shiftc_sc[...]
        o_ref[t] = jnp.maximum(z + x, 0.0).astype(o_ref.dtype)


@jax.jit
def _external_attention(x_nchw, w1, b1, w0, wc2, bn_scale, bn_shift):
    b, c, h, w = x_nchw.shape
    n = h * w

    x = x_nchw.reshape(b, c, n)
    b1r = b1.reshape(1, c)
    scaler = bn_scale.reshape(1, c)
    shiftr = bn_shift.reshape(1, c)

    out = pl.pallas_call(
        _ea_fused_kernel,
        out_shape=jax.ShapeDtypeStruct((b, c, n), x.dtype),
        grid=(b // 4,),
        in_specs=[
            pl.BlockSpec((4, c, n), lambda i: (i, 0, 0)),
            pl.BlockSpec((c, c), lambda i: (0, 0),
                         pipeline_mode=pl.Buffered(buffer_count=1)),
            pl.BlockSpec((1, c), lambda i: (0, 0),
                         pipeline_mode=pl.Buffered(buffer_count=1)),
            pl.BlockSpec((_K, c), lambda i: (0, 0),
                         pipeline_mode=pl.Buffered(buffer_count=1)),
            pl.BlockSpec((c, c), lambda i: (0, 0),
                         pipeline_mode=pl.Buffered(buffer_count=1)),
            pl.BlockSpec((1, c), lambda i: (0, 0),
                         pipeline_mode=pl.Buffered(buffer_count=1)),
            pl.BlockSpec((1, c), lambda i: (0, 0),
                         pipeline_mode=pl.Buffered(buffer_count=1)),
        ],
        out_specs=pl.BlockSpec((4, c, n), lambda i: (i, 0, 0)),
        scratch_shapes=[
            pltpu.VMEM((c, 1), jnp.float32),
            pltpu.VMEM((c, _K), jnp.float32),
            pltpu.VMEM((c, 1), jnp.float32),
        ],
        compiler_params=pltpu.CompilerParams(
            dimension_semantics=("arbitrary",)),
    )(x, w1, b1r, w0, wc2, scaler, shiftr)

    return out.reshape(b, c, h, w)


def kernel(x, w1, b1, w0, wc2, bn_scale, bn_shift):
    return _external_attention(x, w1, b1, w0, wc2, bn_scale, bn_shift)
